# baseline (speedup 1.0000x reference)
import functools

import jax
import jax.numpy as jnp
from jax import lax
from jax.experimental import pallas as pl
from jax.experimental.pallas import tpu as pltpu
from jax.experimental.pallas import tpu_sc as plsc

N_ROWS = 65536
N_CHAN = 256
N_VCHAN = 256
K_TOP = 50
DELTA = 1.0e-3

NC = 2
NS = 16
NW = NC * NS
L = 16

RPW = N_ROWS // NW
CHUNK_ROWS = 128
N_CHUNKS = RPW // CHUNK_ROWS
CHUNK_ELEMS = CHUNK_ROWS * N_CHAN
CAND_PER_W = 64
N_CAND = NW * CAND_PER_W
F32_INF = float("inf")
I32_BIG = 1 << 30

_mesh = plsc.VectorSubcoreMesh(core_axis_name="c", subcore_axis_name="s")
_sc_params = pltpu.CompilerParams(
    needs_layout_passes=False, use_tc_tiling_on_sc=False)


def _iota16():
    return lax.iota(jnp.int32, L)


def _extract_topk(d_ref, n_vregs, emit):
    lane0 = _iota16() == 0

    def scan_body(j, carry):
        macc, jacc, cur = carry
        v = d_ref[pl.ds(j * L, L)]
        m = v < macc
        macc = jnp.where(m, v, macc)
        jacc = jnp.where(m, cur, jacc)
        return macc, jacc, cur + L

    def pass_body(t, _):
        init = (jnp.full((L,), F32_INF), jnp.zeros((L,), jnp.int32), _iota16())
        macc, jacc, _ = lax.fori_loop(0, n_vregs, scan_body, init, unroll=8)
        gmin = jnp.min(macc)
        hit = macc == gmin
        flat = jnp.min(jnp.where(hit, jacc, I32_BIG))
        fvec = jnp.full((L,), flat, jnp.int32)
        emit(t, gmin, fvec)
        plsc.store_scatter(d_ref, [fvec], jnp.full((L,), F32_INF), mask=lane0)
        return 0

    lax.fori_loop(0, K_TOP, pass_body, 0)


def _dnd_stage1(key_hbm, keysf_hbm, cd_out, ci_out, ws_out,
                qv, kbuf, dv, cd_local, ci_local, wtmp,
                sem0, sem1):
    wid = lax.axis_index("s") * NC + lax.axis_index("c")
    base_row = wid * RPW
    iota = _iota16()
    lane0 = iota == 0
    sems = (sem0, sem1)

    pltpu.sync_copy(key_hbm, qv)

    def start_dma(cnk):
        b = cnk % 2
        src = keysf_hbm.at[pl.ds((base_row + cnk * CHUNK_ROWS) * N_CHAN,
                                 CHUNK_ELEMS)]
        return pltpu.async_copy(src,
                                kbuf.at[pl.ds(b * CHUNK_ELEMS, CHUNK_ELEMS)],
                                sems[b])

    iota_row = iota * N_CHAN

    def compute_chunk(cnk, wacc):
        b = cnk % 2

        def g_body(g, wacc):
            fi0 = jnp.full((L,), g * (L * N_CHAN), jnp.int32) + iota_row

            def cg_body(cg, carry):
                acc, fi = carry
                qvec = qv[pl.ds(cg * L, L)]
                for c in range(L):
                    kv = plsc.load_gather(kbuf, [fi + (b * CHUNK_ELEMS + c)])
                    t = kv - qvec[c]
                    acc = acc + t * t
                return acc, fi + L

            acc, _ = lax.fori_loop(0, N_CHAN // L, cg_body,
                                   (jnp.zeros((L,), jnp.float32), fi0))
            dv[pl.ds(cnk * CHUNK_ROWS + g * L, L)] = acc
            return wacc + 1.0 / (acc + DELTA)

        return lax.fori_loop(0, CHUNK_ROWS // L, g_body, wacc)

    copies = [None, None]
    copies[0] = start_dma(0)
    wacc = jnp.zeros((L,), jnp.float32)
    for cnk in range(N_CHUNKS):
        if cnk + 1 < N_CHUNKS:
            copies[(cnk + 1) % 2] = start_dma(cnk + 1)
        copies[cnk % 2].wait()
        wacc = compute_chunk(cnk, wacc)

    wtmp[...] = wacc
    pltpu.sync_copy(wtmp, ws_out.at[pl.ds(wid * L, L)])

    zeros_f = jnp.zeros((L,), jnp.float32)
    zeros_i = jnp.zeros((L,), jnp.int32)
    for s in range(CAND_PER_W // L):
        cd_local[pl.ds(s * L, L)] = zeros_f + F32_INF
        ci_local[pl.ds(s * L, L)] = zeros_i

    gbase = base_row

    def emit(t, gmin, fvec):
        tvec = jnp.full((L,), t, jnp.int32)
        plsc.store_scatter(cd_local, [tvec], jnp.full((L,), gmin), mask=lane0)
        plsc.store_scatter(ci_local, [tvec], fvec + gbase, mask=lane0)

    _extract_topk(dv, RPW // L, emit)

    pltpu.sync_copy(cd_local, cd_out.at[pl.ds(wid * CAND_PER_W, CAND_PER_W)])
    pltpu.sync_copy(ci_local, ci_out.at[pl.ds(wid * CAND_PER_W, CAND_PER_W)])


def _dnd_stage2(cd_hbm, ci_hbm, ws_hbm, values_hbm, out_hbm,
                cdv, civ, wsv, selw, seli, rows, otmp, sem0):
    core = lax.axis_index("c")
    sub = lax.axis_index("s")
    iota = _iota16()
    lane0 = iota == 0

    @pl.when(core == 0)
    def _():
        pltpu.sync_copy(cd_hbm, cdv)
        pltpu.sync_copy(ci_hbm, civ)
        pltpu.sync_copy(ws_hbm, wsv)

        def s_body(i, acc):
            return acc + wsv[pl.ds(i * L, L)]
        ssum = jnp.sum(lax.fori_loop(0, (NW * L) // L, s_body,
                                     jnp.zeros((L,), jnp.float32)))

        zeros_f = jnp.zeros((L,), jnp.float32)
        zeros_i = jnp.zeros((L,), jnp.int32)
        for s in range(CAND_PER_W // L):
            selw[pl.ds(s * L, L)] = zeros_f
            seli[pl.ds(s * L, L)] = zeros_i

        def emit(t, gmin, fvec):
            tvec = jnp.full((L,), t, jnp.int32)
            gidx = plsc.load_gather(civ, [fvec])
            wv = 1.0 / (jnp.full((L,), gmin) + DELTA)
            plsc.store_scatter(selw, [tvec], wv, mask=lane0)
            plsc.store_scatter(seli, [tvec], gidx, mask=lane0)

        _extract_topk(cdv, N_CAND // L, emit)

        pltpu.async_copy(values_hbm.at[seli], rows, sem0).wait()

        ch0 = sub * L

        def r_body(r, acc):
            rvec = jnp.full((L,), r, jnp.int32)
            wv = plsc.load_gather(selw, [rvec])
            rv = plsc.load_gather(rows, [rvec, ch0 + iota])
            return acc + wv * rv

        acc = lax.fori_loop(0, CAND_PER_W, r_body,
                            jnp.zeros((L,), jnp.float32), unroll=4)
        otmp[...] = acc / jnp.full((L,), ssum)
        pltpu.sync_copy(otmp, out_hbm.at[pl.ds(ch0, L)])


_stage1 = pl.kernel(
    _dnd_stage1,
    out_type=(
        jax.ShapeDtypeStruct((N_CAND,), jnp.float32),
        jax.ShapeDtypeStruct((N_CAND,), jnp.int32),
        jax.ShapeDtypeStruct((NW * L,), jnp.float32),
    ),
    mesh=_mesh,
    scratch_types=[
        pltpu.VMEM((N_CHAN,), jnp.float32),
        pltpu.VMEM((2 * CHUNK_ELEMS,), jnp.float32),
        pltpu.VMEM((RPW,), jnp.float32),
        pltpu.VMEM((CAND_PER_W,), jnp.float32),
        pltpu.VMEM((CAND_PER_W,), jnp.int32),
        pltpu.VMEM((L,), jnp.float32),
        pltpu.SemaphoreType.DMA,
        pltpu.SemaphoreType.DMA,
    ],
    compiler_params=_sc_params,
)

_stage2 = pl.kernel(
    _dnd_stage2,
    out_type=jax.ShapeDtypeStruct((N_VCHAN,), jnp.float32),
    mesh=_mesh,
    scratch_types=[
        pltpu.VMEM((N_CAND,), jnp.float32),
        pltpu.VMEM((N_CAND,), jnp.int32),
        pltpu.VMEM((NW * L,), jnp.float32),
        pltpu.VMEM((CAND_PER_W,), jnp.float32),
        pltpu.VMEM((CAND_PER_W,), jnp.int32),
        pltpu.VMEM((CAND_PER_W, N_VCHAN), jnp.float32),
        pltpu.VMEM((L,), jnp.float32),
        pltpu.SemaphoreType.DMA,
    ],
    compiler_params=_sc_params,
)


@jax.jit
def kernel(key, keys, values):
    cand_d, cand_i, wsums = _stage1(key, keys.reshape(-1))
    out = _stage2(cand_d, cand_i, wsums, values)
    return out.reshape(1, N_VCHAN)


# speedup vs baseline: 4.1189x; 4.1189x over previous
import functools

import jax
import jax.numpy as jnp
from jax import lax
from jax.experimental import pallas as pl
from jax.experimental.pallas import tpu as pltpu
from jax.experimental.pallas import tpu_sc as plsc

N_ROWS = 65536
N_CHAN = 256
N_VCHAN = 256
K_TOP = 50
DELTA = 1.0e-3

NC = 2
NS = 16
NW = NC * NS
L = 16

RPW = N_ROWS // NW
CHUNK_ROWS = 128
N_CHUNKS = RPW // CHUNK_ROWS
CHUNK_ELEMS = CHUNK_ROWS * N_CHAN
CAND_PER_W = 64
N_CAND = NW * CAND_PER_W
GATHER_PER_STEP = 8
F32_INF = float("inf")
I32_BIG = 1 << 30

_mesh = plsc.VectorSubcoreMesh(core_axis_name="c", subcore_axis_name="s")
_sc_params = pltpu.CompilerParams(
    needs_layout_passes=False, use_tc_tiling_on_sc=False)


def _iota16():
    return lax.iota(jnp.int32, L)


def _tree_sum(terms):
    while len(terms) > 1:
        nxt = [terms[i] + terms[i + 1] for i in range(0, len(terms) - 1, 2)]
        if len(terms) % 2:
            nxt.append(terms[-1])
        terms = nxt
    return terms[0]


def _extract_topk(d_ref, n_vregs, emit):
    lane0 = _iota16() == 0

    def scan_body(j, carry):
        macc, jacc, cur = carry
        v = d_ref[pl.ds(j * L, L)]
        m = v < macc
        macc = jnp.where(m, v, macc)
        jacc = jnp.where(m, cur, jacc)
        return macc, jacc, cur + L

    def pass_body(t, _):
        init = (jnp.full((L,), F32_INF), jnp.zeros((L,), jnp.int32), _iota16())
        macc, jacc, _ = lax.fori_loop(0, n_vregs, scan_body, init, unroll=8)
        gmin = jnp.min(macc)
        hit = macc == gmin
        flat = jnp.min(jnp.where(hit, jacc, I32_BIG))
        fvec = jnp.full((L,), flat, jnp.int32)
        emit(t, gmin, fvec)
        plsc.store_scatter(d_ref, [fvec], jnp.full((L,), F32_INF), mask=lane0)
        return 0

    lax.fori_loop(0, K_TOP, pass_body, 0)


def _dnd_stage1(key_hbm, keysf_hbm, cd_out, ci_out, ws_out,
                qv, kbuf, dv, cd_local, ci_local, wtmp,
                sem0, sem1):
    wid = lax.axis_index("s") * NC + lax.axis_index("c")
    base_row = wid * RPW
    iota = _iota16()
    lane0 = iota == 0
    sems = (sem0, sem1)

    pltpu.sync_copy(key_hbm, qv)
    qregs = [qv[pl.ds(cs * L, L)] for cs in range(N_CHAN // L)]

    nblk = CHUNK_ROWS // 8

    def start_dma(cnk):
        b = cnk % 2
        blk0 = (base_row + cnk * CHUNK_ROWS) // 8
        return pltpu.async_copy(keysf_hbm.at[pl.ds(blk0, nblk)],
                                kbuf.at[pl.ds(b * nblk, nblk)],
                                sems[b])

    def compute_chunk(cnk, wacc):
        b = cnk % 2

        def g_body(g, wacc):
            dvv = jnp.zeros((L,), jnp.float32)
            for rbb in range(2):
                blk = b * nblk + 2 * g + rbb
                for rr in range(8):
                    terms = []
                    for ct in range(2):
                        for cs in range(8):
                            kv = kbuf[blk, ct, rr, pl.ds(cs * L, L)]
                            t = kv - qregs[ct * 8 + cs]
                            terms.append(t * t)
                    dr = jnp.sum(_tree_sum(terms))
                    dvv = jnp.where(iota == rbb * 8 + rr,
                                    jnp.full((L,), dr), dvv)
            dv[pl.ds(cnk * CHUNK_ROWS + g * L, L)] = dvv
            return wacc + 1.0 / (dvv + DELTA)

        return lax.fori_loop(0, CHUNK_ROWS // L, g_body, wacc)

    copies = [None, None]
    copies[0] = start_dma(0)
    wacc = jnp.zeros((L,), jnp.float32)
    for cnk in range(N_CHUNKS):
        if cnk + 1 < N_CHUNKS:
            copies[(cnk + 1) % 2] = start_dma(cnk + 1)
        copies[cnk % 2].wait()
        wacc = compute_chunk(cnk, wacc)

    wtmp[...] = wacc
    pltpu.sync_copy(wtmp, ws_out.at[pl.ds(wid * L, L)])

    zeros_f = jnp.zeros((L,), jnp.float32)
    zeros_i = jnp.zeros((L,), jnp.int32)
    for s in range(CAND_PER_W // L):
        cd_local[pl.ds(s * L, L)] = zeros_f + F32_INF
        ci_local[pl.ds(s * L, L)] = zeros_i

    gbase = base_row

    def emit(t, gmin, fvec):
        tvec = jnp.full((L,), t, jnp.int32)
        plsc.store_scatter(cd_local, [tvec], jnp.full((L,), gmin), mask=lane0)
        plsc.store_scatter(ci_local, [tvec], fvec + gbase, mask=lane0)

    _extract_topk(dv, RPW // L, emit)

    pltpu.sync_copy(cd_local, cd_out.at[pl.ds(wid * CAND_PER_W, CAND_PER_W)])
    pltpu.sync_copy(ci_local, ci_out.at[pl.ds(wid * CAND_PER_W, CAND_PER_W)])


def _dnd_stage2(cd_hbm, ci_hbm, ws_hbm, sw_out, si_out,
                cdv, civ, wsv, selw, seli):
    core = lax.axis_index("c")
    sub = lax.axis_index("s")
    iota = _iota16()
    lane0 = iota == 0

    @pl.when((core == 0) & (sub == 0))
    def _():
        pltpu.sync_copy(cd_hbm, cdv)
        pltpu.sync_copy(ci_hbm, civ)
        pltpu.sync_copy(ws_hbm, wsv)

        def s_body(i, acc):
            return acc + wsv[pl.ds(i * L, L)]
        ssum = jnp.sum(lax.fori_loop(0, (NW * L) // L, s_body,
                                     jnp.zeros((L,), jnp.float32)))
        svec = jnp.full((L,), ssum)

        zeros_f = jnp.zeros((L,), jnp.float32)
        zeros_i = jnp.zeros((L,), jnp.int32)
        for s in range(CAND_PER_W // L):
            selw[pl.ds(s * L, L)] = zeros_f
            seli[pl.ds(s * L, L)] = zeros_i

        def emit(t, gmin, fvec):
            tvec = jnp.full((L,), t, jnp.int32)
            gidx = plsc.load_gather(civ, [fvec])
            wv = 1.0 / ((jnp.full((L,), gmin) + DELTA) * svec)
            plsc.store_scatter(selw, [tvec], wv, mask=lane0)
            plsc.store_scatter(seli, [tvec], gidx, mask=lane0)

        _extract_topk(cdv, N_CAND // L, emit)

        pltpu.sync_copy(selw, sw_out)
        pltpu.sync_copy(seli, si_out)


def _dnd_stage3(si_smem, sw_smem, *refs):
    vrefs = refs[:GATHER_PER_STEP]
    out_ref = refs[GATHER_PER_STEP]
    i = pl.program_id(0)

    @pl.when(i == 0)
    def _():
        out_ref[...] = jnp.zeros_like(out_ref)

    acc = out_ref[...]
    for j in range(GATHER_PER_STEP):
        k = i * GATHER_PER_STEP + j
        rr = si_smem[k] % 8
        blk = vrefs[j][...]
        sel = lax.broadcasted_iota(jnp.int32, (8, N_VCHAN), 0) == rr
        row = jnp.sum(jnp.where(sel, blk, 0.0), axis=0, keepdims=True)
        acc = acc + row * sw_smem[k]
    out_ref[...] = acc


_stage1 = pl.kernel(
    _dnd_stage1,
    out_type=(
        jax.ShapeDtypeStruct((N_CAND,), jnp.float32),
        jax.ShapeDtypeStruct((N_CAND,), jnp.int32),
        jax.ShapeDtypeStruct((NW * L,), jnp.float32),
    ),
    mesh=_mesh,
    scratch_types=[
        pltpu.VMEM((N_CHAN,), jnp.float32),
        pltpu.VMEM((2 * CHUNK_ROWS // 8, 2, 8, 128), jnp.float32),
        pltpu.VMEM((RPW,), jnp.float32),
        pltpu.VMEM((CAND_PER_W,), jnp.float32),
        pltpu.VMEM((CAND_PER_W,), jnp.int32),
        pltpu.VMEM((L,), jnp.float32),
        pltpu.SemaphoreType.DMA,
        pltpu.SemaphoreType.DMA,
    ],
    compiler_params=_sc_params,
)

_stage2 = pl.kernel(
    _dnd_stage2,
    out_type=(
        jax.ShapeDtypeStruct((CAND_PER_W,), jnp.float32),
        jax.ShapeDtypeStruct((CAND_PER_W,), jnp.int32),
    ),
    mesh=_mesh,
    scratch_types=[
        pltpu.VMEM((N_CAND,), jnp.float32),
        pltpu.VMEM((N_CAND,), jnp.int32),
        pltpu.VMEM((NW * L,), jnp.float32),
        pltpu.VMEM((CAND_PER_W,), jnp.float32),
        pltpu.VMEM((CAND_PER_W,), jnp.int32),
    ],
    compiler_params=_sc_params,
)


def _mk_value_spec(j):
    return pl.BlockSpec(
        (8, N_VCHAN),
        lambda i, si, sw, j=j: (si[i * GATHER_PER_STEP + j] // 8, 0))


_stage3 = pl.pallas_call(
    _dnd_stage3,
    grid_spec=pltpu.PrefetchScalarGridSpec(
        num_scalar_prefetch=2,
        grid=(CAND_PER_W // GATHER_PER_STEP,),
        in_specs=[_mk_value_spec(j) for j in range(GATHER_PER_STEP)],
        out_specs=pl.BlockSpec((1, N_VCHAN), lambda i, si, sw: (0, 0)),
    ),
    out_shape=jax.ShapeDtypeStruct((1, N_VCHAN), jnp.float32),
    compiler_params=pltpu.CompilerParams(
        dimension_semantics=("arbitrary",)),
)


@jax.jit
def kernel(key, keys, values):
    keys4 = keys.reshape(N_ROWS // 8, 8, 2, 128).transpose(0, 2, 1, 3)
    cand_d, cand_i, wsums = _stage1(key, keys4)
    selw, seli = _stage2(cand_d, cand_i, wsums)
    return _stage3(seli, selw, *([values] * GATHER_PER_STEP))
